# baseline (speedup 1.0000x reference)
import functools

import jax
import jax.numpy as jnp
from jax import lax
from jax.experimental import pallas as pl
from jax.experimental.pallas import tpu as pltpu
from jax.experimental.pallas import tpu_sc as plsc

_L = 128
_NB = 4


def _make_gather_t(S1, S0, D, NW, R):
    G = R // _L
    n_ch = S0 // R
    n_tasks = S1 * n_ch
    per_w = n_tasks // NW
    assert n_tasks % NW == 0
    assert per_w % _NB == 0 and per_w >= 2 * _NB
    mesh = plsc.VectorSubcoreMesh(core_axis_name="c", subcore_axis_name="s")

    @functools.partial(
        pl.kernel,
        mesh=mesh,
        out_type=jax.ShapeDtypeStruct((S1, D, S0), jnp.float32),
        scratch_types=[
            pltpu.VMEM((_NB, G, _L), jnp.int32),
            pltpu.VMEM((_NB, R, D), jnp.float32),
            pltpu.VMEM((2, D, R), jnp.float32),
            [pltpu.SemaphoreType.DMA] * _NB,
            [pltpu.SemaphoreType.DMA] * 2,
        ],
        compiler_params=pltpu.CompilerParams(
            use_tc_tiling_on_sc=False, needs_layout_passes=False),
    )
    def gather_kernel(t2_hbm, table_hbm, out_hbm, idx_v, rows_v, tr_v,
                      sg, so):
        wid = lax.axis_index("s") * 2 + lax.axis_index("c")
        iota16 = lax.iota(jnp.int32, 16)
        diag = [(c + lax.iota(jnp.int32, 16)) % D for c in range(D)]

        def stage_and_fire(task, b):
            s1 = task // n_ch
            ch = task % n_ch
            pltpu.sync_copy(t2_hbm.at[s1, pl.ds(ch * G, G)], idx_v.at[b])
            for j in range(G):
                pltpu.make_async_copy(
                    table_hbm.at[idx_v.at[b, j]],
                    rows_v.at[b, pl.ds(j * _L, _L)],
                    sg[b],
                ).start()

        def wait_gathers(b):
            for j in range(G):
                pltpu.make_async_copy(
                    table_hbm.at[idx_v.at[b, j]],
                    rows_v.at[b, pl.ds(j * _L, _L)],
                    sg[b],
                ).wait()

        def transpose(b, tb):
            def t_body(t, __):
                row = t * 16 + iota16
                for c0 in range(0, D, 8):
                    vs = [plsc.load_gather(rows_v.at[b], [row, diag[c]])
                          for c in range(c0, c0 + 8)]
                    for k, c in enumerate(range(c0, c0 + 8)):
                        plsc.store_scatter(tr_v.at[tb], [diag[c], row], vs[k])
                return __

            lax.fori_loop(0, R // 16, t_body, 0)

        def out_copy(task, tb):
            s1 = task // n_ch
            ch = task % n_ch
            return pltpu.make_async_copy(
                tr_v.at[tb], out_hbm.at[s1, :, pl.ds(ch * R, R)], so[tb])

        def out_start(task, tb):
            out_copy(task, tb).start()

        def out_wait(task, tb):
            out_copy(task, tb).wait()

        base = wid * per_w
        for b in range(_NB):
            stage_and_fire(base + b, b)

        for b in range(_NB):
            g = base + b
            tb = b % 2
            wait_gathers(b)
            if b >= 2:
                out_wait(g - 2, tb)
            transpose(b, tb)
            out_start(g, tb)
            stage_and_fire(g + _NB, b)

        def group_body(p, carry):
            g0 = base + _NB * p
            for b in range(_NB):
                g = g0 + b
                tb = b % 2
                wait_gathers(b)
                out_wait(g - 2, tb)
                transpose(b, tb)
                out_start(g, tb)
                stage_and_fire(g + _NB, b)
            return carry

        lax.fori_loop(1, per_w // _NB - 1, group_body, 0)

        for b in range(_NB):
            g = base + per_w - _NB + b
            tb = b % 2
            wait_gathers(b)
            out_wait(g - 2, tb)
            transpose(b, tb)
            out_start(g, tb)
        for b in (0, 1):
            out_wait(base + per_w - 2 + b, b)

    return gather_kernel


def kernel(token_ids, weight):
    S0, S1 = token_ids.shape
    V, D = weight.shape
    NW = 32
    R = 256
    H = S1 // 2
    assert S0 % _L == 0 and (H * (S0 // R)) % NW == 0
    t2r = token_ids.T.reshape(S1, S0 // _L, _L)
    gk = _make_gather_t(H, S0, D, NW, R)
    halves = [gk(t2r[h * H:(h + 1) * H], weight) for h in range(2)]
    outs = [h_out.transpose(2, 0, 1) for h_out in halves]
    return jnp.concatenate(outs, axis=1)


# speedup vs baseline: 1.1164x; 1.1164x over previous
import functools

import jax
import jax.numpy as jnp
from jax import lax
from jax.experimental import pallas as pl
from jax.experimental.pallas import tpu as pltpu
from jax.experimental.pallas import tpu_sc as plsc

_L = 128
_NB = 4


def _make_gather_t(S1, S0, D, NW, R):
    G = R // _L
    n_ch = S0 // R
    n_tasks = S1 * n_ch
    per_w = n_tasks // NW
    assert n_tasks % NW == 0
    assert per_w % _NB == 0 and per_w >= 2 * _NB
    mesh = plsc.VectorSubcoreMesh(core_axis_name="c", subcore_axis_name="s")

    @functools.partial(
        pl.kernel,
        mesh=mesh,
        out_type=jax.ShapeDtypeStruct((S1, D, S0), jnp.float32),
        scratch_types=[
            pltpu.VMEM((_NB, G, _L), jnp.int32),
            pltpu.VMEM((_NB, R, D), jnp.float32),
            pltpu.VMEM((2, D, R), jnp.float32),
            [pltpu.SemaphoreType.DMA] * _NB,
            [pltpu.SemaphoreType.DMA] * 2,
        ],
        compiler_params=pltpu.CompilerParams(
            use_tc_tiling_on_sc=False, needs_layout_passes=False),
    )
    def gather_kernel(t2_hbm, table_hbm, out_hbm, idx_v, rows_v, tr_v,
                      sg, so):
        wid = lax.axis_index("s") * 2 + lax.axis_index("c")
        iota16 = lax.iota(jnp.int32, 16)
        diag = [(c + lax.iota(jnp.int32, 16)) % D for c in range(D)]

        def stage_and_fire(task, b):
            s1 = task // n_ch
            ch = task % n_ch
            pltpu.sync_copy(t2_hbm.at[s1, pl.ds(ch * G, G)], idx_v.at[b])
            for j in range(G):
                pltpu.make_async_copy(
                    table_hbm.at[idx_v.at[b, j]],
                    rows_v.at[b, pl.ds(j * _L, _L)],
                    sg[b],
                ).start()

        def wait_gathers(b):
            for j in range(G):
                pltpu.make_async_copy(
                    table_hbm.at[idx_v.at[b, j]],
                    rows_v.at[b, pl.ds(j * _L, _L)],
                    sg[b],
                ).wait()

        def transpose(b, tb):
            def t_body(t, __):
                row = t * 16 + iota16
                for c0 in range(0, D, 16):
                    vs = [plsc.load_gather(rows_v.at[b], [row, diag[c]])
                          for c in range(c0, c0 + 16)]
                    for k, c in enumerate(range(c0, c0 + 16)):
                        plsc.store_scatter(tr_v.at[tb], [diag[c], row], vs[k])
                return __

            lax.fori_loop(0, R // 16, t_body, 0)

        def out_copy(task, tb):
            s1 = task // n_ch
            ch = task % n_ch
            return pltpu.make_async_copy(
                tr_v.at[tb], out_hbm.at[s1, :, pl.ds(ch * R, R)], so[tb])

        def out_start(task, tb):
            out_copy(task, tb).start()

        def out_wait(task, tb):
            out_copy(task, tb).wait()

        base = wid * per_w
        for b in range(_NB):
            stage_and_fire(base + b, b)

        for b in range(_NB):
            g = base + b
            tb = b % 2
            wait_gathers(b)
            if b >= 2:
                out_wait(g - 2, tb)
            transpose(b, tb)
            out_start(g, tb)
            stage_and_fire(g + _NB, b)

        def group_body(p, carry):
            g0 = base + _NB * p
            for b in range(_NB):
                g = g0 + b
                tb = b % 2
                wait_gathers(b)
                out_wait(g - 2, tb)
                transpose(b, tb)
                out_start(g, tb)
                stage_and_fire(g + _NB, b)
            return carry

        lax.fori_loop(1, per_w // _NB - 1, group_body, 0)

        for b in range(_NB):
            g = base + per_w - _NB + b
            tb = b % 2
            wait_gathers(b)
            out_wait(g - 2, tb)
            transpose(b, tb)
            out_start(g, tb)
        for b in (0, 1):
            out_wait(base + per_w - 2 + b, b)

    return gather_kernel


def kernel(token_ids, weight):
    S0, S1 = token_ids.shape
    V, D = weight.shape
    NW = 32
    R = 512
    assert S0 % _L == 0 and (S1 * (S0 // R)) % NW == 0
    t2r = token_ids.T.reshape(S1, S0 // _L, _L)
    out_p = _make_gather_t(S1, S0, D, NW, R)(t2r, weight)
    return out_p.transpose(2, 0, 1)


# speedup vs baseline: 1.1188x; 1.0021x over previous
import functools

import jax
import jax.numpy as jnp
from jax import lax
from jax.experimental import pallas as pl
from jax.experimental.pallas import tpu as pltpu
from jax.experimental.pallas import tpu_sc as plsc

_L = 128
_NB = 4


def _make_gather_t(S1, S0, D, NW, R):
    G = R // _L
    n_ch = S0 // R
    n_tasks = S1 * n_ch
    per_w = n_tasks // NW
    assert n_tasks % NW == 0
    assert per_w % _NB == 0 and per_w >= 2 * _NB
    mesh = plsc.VectorSubcoreMesh(core_axis_name="c", subcore_axis_name="s")

    @functools.partial(
        pl.kernel,
        mesh=mesh,
        out_type=jax.ShapeDtypeStruct((S1, D, S0), jnp.float32),
        scratch_types=[
            pltpu.VMEM((_NB, G, _L), jnp.int32),
            pltpu.VMEM((_NB, R, D), jnp.float32),
            pltpu.VMEM((2, D, R), jnp.float32),
            [pltpu.SemaphoreType.DMA] * _NB,
            [pltpu.SemaphoreType.DMA] * 2,
        ],
        compiler_params=pltpu.CompilerParams(
            use_tc_tiling_on_sc=False, needs_layout_passes=False),
    )
    def gather_kernel(t2_hbm, table_hbm, out_hbm, idx_v, rows_v, tr_v,
                      sg, so):
        wid = lax.axis_index("s") * 2 + lax.axis_index("c")
        iota16 = lax.iota(jnp.int32, 16)
        diag = [(c + lax.iota(jnp.int32, 16)) % D for c in range(D)]

        def stage_and_fire(task, b):
            s1 = task // n_ch
            ch = task % n_ch
            pltpu.sync_copy(t2_hbm.at[s1, pl.ds(ch * G, G)], idx_v.at[b])
            for j in range(G):
                pltpu.make_async_copy(
                    table_hbm.at[idx_v.at[b, j]],
                    rows_v.at[b, pl.ds(j * _L, _L)],
                    sg[b],
                ).start()

        def wait_gathers(b):
            for j in range(G):
                pltpu.make_async_copy(
                    table_hbm.at[idx_v.at[b, j]],
                    rows_v.at[b, pl.ds(j * _L, _L)],
                    sg[b],
                ).wait()

        def transpose(b, tb):
            def t_body(t, __):
                row = t * 16 + iota16
                for c0 in range(0, D, 8):
                    vs = [plsc.load_gather(rows_v.at[b], [row, diag[c]])
                          for c in range(c0, c0 + 8)]
                    for k, c in enumerate(range(c0, c0 + 8)):
                        plsc.store_scatter(tr_v.at[tb], [diag[c], row], vs[k])
                return __

            lax.fori_loop(0, R // 16, t_body, 0)

        def out_copy(task, tb):
            s1 = task // n_ch
            ch = task % n_ch
            return pltpu.make_async_copy(
                tr_v.at[tb], out_hbm.at[s1, :, pl.ds(ch * R, R)], so[tb])

        def out_start(task, tb):
            out_copy(task, tb).start()

        def out_wait(task, tb):
            out_copy(task, tb).wait()

        base = wid * per_w
        for b in range(_NB):
            stage_and_fire(base + b, b)

        for b in range(_NB):
            g = base + b
            tb = b % 2
            wait_gathers(b)
            if b >= 2:
                out_wait(g - 2, tb)
            transpose(b, tb)
            out_start(g, tb)
            stage_and_fire(g + _NB, b)

        def group_body(p, carry):
            g0 = base + _NB * p
            for b in range(_NB):
                g = g0 + b
                tb = b % 2
                wait_gathers(b)
                out_wait(g - 2, tb)
                transpose(b, tb)
                out_start(g, tb)
                stage_and_fire(g + _NB, b)
            return carry

        lax.fori_loop(1, per_w // _NB - 1, group_body, 0)

        for b in range(_NB):
            g = base + per_w - _NB + b
            tb = b % 2
            wait_gathers(b)
            out_wait(g - 2, tb)
            transpose(b, tb)
            out_start(g, tb)
        for b in (0, 1):
            out_wait(base + per_w - 2 + b, b)

    return gather_kernel


def kernel(token_ids, weight):
    S0, S1 = token_ids.shape
    V, D = weight.shape
    NW = 32
    R = 512
    assert S0 % _L == 0 and (S1 * (S0 // R)) % NW == 0
    t2r = token_ids.T.reshape(S1, S0 // _L, _L)
    out_p = _make_gather_t(S1, S0, D, NW, R)(t2r, weight)
    return out_p.transpose(2, 0, 1)
